# Initial kernel scaffold; baseline (speedup 1.0000x reference)
import jax
import jax.numpy as jnp
from jax.experimental import pallas as pl


def kernel(x, edge_index, W, b):
    raise NotImplementedError("write your pallas kernel here")


# speedup vs baseline: 24.4184x; 24.4184x over previous
import jax
import jax.numpy as jnp
from jax import lax
from jax.experimental import pallas as pl
from jax.experimental.pallas import tpu as pltpu
from jax.experimental.pallas import tpu_sc as plsc

N_NODES = 10000
N_EDGES = 320000
D_FEAT = 128

NC = 2
NS = 16
NW = NC * NS

NP = 10240
ROWS_W = NP // NW
ROWS_T = NP // NS

E_TILE = N_EDGES // NS
CH = 125
KCH = N_EDGES // CH
K_W = KCH // NW

_mesh = plsc.VectorSubcoreMesh(core_axis_name="c", subcore_axis_name="s")


def _rsqrt_newton(v):
  i = plsc.bitcast(v, jnp.int32)
  y = plsc.bitcast(jnp.full((16,), 0x5F3759DF, jnp.int32)
                   - lax.shift_right_logical(i, 1), jnp.float32)
  half = v * 0.5
  for _ in range(3):
    y = y * (1.5 - half * y * y)
  return jnp.where(v > 0.5, y, jnp.zeros((16,), jnp.float32))


def _deg_scale_body(row_hbm, xp_hbm, xs_hbm, dis_hbm,
                    row_v, deg_v, acc_v, tmp_v, x_v,
                    deg_all):
  c = lax.axis_index("c")
  t = lax.axis_index("s")
  w = t * NC + c

  def _z(i, _):
    deg_v[pl.ds(i * 16, 16)] = jnp.zeros((16,), jnp.float32)
    return 0
  lax.fori_loop(0, NP // 16, _z, 0)

  pltpu.sync_copy(row_hbm.at[pl.ds(t * E_TILE, E_TILE)], row_v)
  ones = jnp.ones((16,), jnp.float32)

  def _hist(i, _):
    idx = row_v[pl.ds(i * 16, 16)]
    plsc.addupdate_scatter(deg_v, [idx], ones)
    return 0
  lax.fori_loop(0, E_TILE // 16, _hist, 0)

  pltpu.sync_copy(deg_v, deg_all.at[t])
  plsc.subcore_barrier()

  pltpu.sync_copy(deg_all.at[0, pl.ds(t * ROWS_T, ROWS_T)], acc_v)
  for j in range(1, NS):
    pltpu.sync_copy(deg_all.at[j, pl.ds(t * ROWS_T, ROWS_T)], tmp_v)

    def _add(i, _):
      acc_v[pl.ds(i * 16, 16)] = (acc_v[pl.ds(i * 16, 16)]
                                  + tmp_v[pl.ds(i * 16, 16)])
      return 0
    lax.fori_loop(0, ROWS_T // 16, _add, 0)

  def _newton(i, _):
    acc_v[pl.ds(i * 16, 16)] = _rsqrt_newton(acc_v[pl.ds(i * 16, 16)])
    return 0
  lax.fori_loop(0, ROWS_T // 16, _newton, 0)
  pltpu.sync_copy(acc_v.at[pl.ds(c * ROWS_W, ROWS_W)],
                  dis_hbm.at[pl.ds(t * ROWS_T + c * ROWS_W, ROWS_W)])

  r0 = t * ROWS_T + c * ROWS_W
  pltpu.sync_copy(xp_hbm.at[pl.ds(r0, ROWS_W)], x_v)

  def _scale(r, _):
    dv = plsc.load_gather(acc_v, [jnp.zeros((16,), jnp.int32) + (r + c * ROWS_W)])
    for j in range(8):
      x_v[r, pl.ds(j * 16, 16)] = x_v[r, pl.ds(j * 16, 16)] * dv
    return 0
  lax.fori_loop(0, ROWS_W, _scale, 0)
  pltpu.sync_copy(x_v, xs_hbm.at[pl.ds(r0, ROWS_W)])


_deg_scale = pl.kernel(
    _deg_scale_body,
    out_type=(jax.ShapeDtypeStruct((NP, D_FEAT), jnp.float32),
              jax.ShapeDtypeStruct((NP,), jnp.float32)),
    mesh=_mesh,
    compiler_params=pltpu.CompilerParams(needs_layout_passes=False),
    scratch_types=(
        pltpu.VMEM((E_TILE,), jnp.int32),
        pltpu.VMEM((NP,), jnp.float32),
        pltpu.VMEM((ROWS_T,), jnp.float32),
        pltpu.VMEM((ROWS_T,), jnp.float32),
        pltpu.VMEM((ROWS_W, D_FEAT), jnp.float32),
        pltpu.VMEM_SHARED((NS, NP), jnp.float32),
    ),
)


K_H = K_W // 2


def _aggregate_body(xs_hbm, col_hbm, row_hbm, aggp_hbm,
                    col_l, row_l, buf0, buf1, sem0, sem1,
                    agg_s):
  c = lax.axis_index("c")
  t = lax.axis_index("s")

  def _zb(i, _):
    def _zb2(j, _2):
      buf0[i, pl.ds(j * 16, 16)] = jnp.zeros((16,), jnp.float32)
      return 0
    lax.fori_loop(0, 8, _zb2, 0)
    return 0
  lax.fori_loop(0, 128, _zb, 0)
  for i in range(5):
    pltpu.sync_copy(buf0, agg_s.at[pl.ds(t * ROWS_T + i * 128, 128)])

  plsc.subcore_barrier()

  base = c * (KCH // NC) + t * K_W
  b0 = buf0.at[pl.ds(0, CH)]
  b1 = buf1.at[pl.ds(0, CH)]
  for h in range(2):
    pltpu.sync_copy(col_hbm.at[pl.ds(base + h * K_H, K_H)], col_l)
    pltpu.sync_copy(row_hbm.at[pl.ds(base + h * K_H, K_H)], row_l)

    pltpu.async_copy(xs_hbm.at[col_l.at[0]], b0, sem0)
    pltpu.async_copy(xs_hbm.at[col_l.at[1]], b1, sem1)

    def _pair(k2, _):
      k = k2 * 2
      pltpu.make_async_copy(xs_hbm.at[col_l.at[k]], b0, sem0).wait()
      pltpu.sync_copy(b0, agg_s.at[row_l.at[k]], add=True)
      pltpu.async_copy(xs_hbm.at[col_l.at[k + 2]], b0, sem0)
      pltpu.make_async_copy(xs_hbm.at[col_l.at[k + 1]], b1, sem1).wait()
      pltpu.sync_copy(b1, agg_s.at[row_l.at[k + 1]], add=True)
      pltpu.async_copy(xs_hbm.at[col_l.at[k + 3]], b1, sem1)
      return 0
    lax.fori_loop(0, K_H // 2 - 1, _pair, 0)

    k = K_H - 2
    pltpu.make_async_copy(xs_hbm.at[col_l.at[k]], b0, sem0).wait()
    pltpu.sync_copy(b0, agg_s.at[row_l.at[k]], add=True)
    pltpu.make_async_copy(xs_hbm.at[col_l.at[k + 1]], b1, sem1).wait()
    pltpu.sync_copy(b1, agg_s.at[row_l.at[k + 1]], add=True)

  plsc.subcore_barrier()

  for i in range(5):
    pltpu.sync_copy(agg_s.at[pl.ds(t * ROWS_T + i * 128, 128)],
                    aggp_hbm.at[c, pl.ds(t * ROWS_T + i * 128, 128)])


_aggregate = pl.kernel(
    _aggregate_body,
    out_type=jax.ShapeDtypeStruct((NC, NP, D_FEAT), jnp.float32),
    mesh=_mesh,
    compiler_params=pltpu.CompilerParams(needs_layout_passes=False),
    scratch_types=(
        pltpu.VMEM((K_H, CH), jnp.int32),
        pltpu.VMEM((K_H, CH), jnp.int32),
        pltpu.VMEM((128, D_FEAT), jnp.float32),
        pltpu.VMEM((128, D_FEAT), jnp.float32),
        pltpu.SemaphoreType.DMA,
        pltpu.SemaphoreType.DMA,
        pltpu.VMEM_SHARED((NP, D_FEAT), jnp.float32),
    ),
)


def _project_body(aggp_ref, dis_ref, w_ref, b_ref, out_ref):
  a = (aggp_ref[0] + aggp_ref[1]) * dis_ref[...]
  y = lax.dot_general(a, w_ref[...], (((1,), (1,)), ((), ())),
                      preferred_element_type=jnp.float32)
  out_ref[...] = jnp.maximum(y + b_ref[...], 0.0)


_BM = 1000


def _project(aggp, dis_col, w, b2):
  return pl.pallas_call(
      _project_body,
      grid=(N_NODES // _BM,),
      in_specs=[
          pl.BlockSpec((NC, _BM, D_FEAT), lambda i: (0, i, 0)),
          pl.BlockSpec((_BM, 1), lambda i: (i, 0)),
          pl.BlockSpec((D_FEAT, D_FEAT), lambda i: (0, 0)),
          pl.BlockSpec((1, D_FEAT), lambda i: (0, 0)),
      ],
      out_specs=pl.BlockSpec((_BM, D_FEAT), lambda i: (i, 0)),
      out_shape=jax.ShapeDtypeStruct((N_NODES, D_FEAT), jnp.float32),
  )(aggp, dis_col, w, b2)


@jax.jit
def kernel(x, edge_index, W, b):
  row = edge_index[0]
  col = edge_index[1]
  xp = jnp.pad(x, ((0, NP - N_NODES), (0, 0)))
  col2d = col.reshape(KCH, CH)
  row2d = row.reshape(KCH, CH)

  xs, dis = _deg_scale(row, xp)
  aggp = _aggregate(xs, col2d, row2d)
  dis_col = dis[:N_NODES].reshape(N_NODES, 1)
  return _project(aggp, dis_col, W, b.reshape(1, D_FEAT))
